# baseline (speedup 1.0000x reference)
import functools

import jax
import jax.numpy as jnp
from jax import lax
from jax.experimental import pallas as pl
from jax.experimental.pallas import tpu as pltpu
from jax.experimental.pallas import tpu_sc as plsc

_N = 10000
_E = 320000
_D = 128
_H = 128
_C = 20
_TEMP = 10.0

_HE = 144
_NC = 2
_NS = 16
_NW = _NC * _NS
_CHUNK = 64
_NCHUNK = 160
_NCHUNKA = _NCHUNK + 1
_EPAD = _NW * _NCHUNK * _CHUNK
_RPS = 632
_AROWS = _NS * _RPS

_BN = 2000


def _pre_body(x_ref, win_ref, bin_ref, wr_ref, bl_ref, fe_ref, base_ref):
    xb = x_ref[...]
    feat = jnp.dot(xb, win_ref[...], preferred_element_type=jnp.float32)
    feat = jnp.maximum(feat + bin_ref[...], 0.0)
    col = lax.broadcasted_iota(jnp.int32, (_BN, _HE - _H), 1)
    tail = jnp.where(col == 0, 1.0, 0.0).astype(jnp.float32)
    fe_ref[...] = jnp.concatenate([feat, tail], axis=1)
    base = jnp.dot(feat, wr_ref[...], preferred_element_type=jnp.float32)
    base_ref[...] = base + bl_ref[...]


def _pre(x, w_in, b_in, w_r, b_l):
    grid = _N // _BN
    return pl.pallas_call(
        _pre_body,
        grid=(grid,),
        in_specs=[
            pl.BlockSpec((_BN, _D), lambda i: (i, 0)),
            pl.BlockSpec((_D, _H), lambda i: (0, 0)),
            pl.BlockSpec((1, _H), lambda i: (0, 0)),
            pl.BlockSpec((_H, _H), lambda i: (0, 0)),
            pl.BlockSpec((1, _H), lambda i: (0, 0)),
        ],
        out_specs=[
            pl.BlockSpec((_BN, _HE), lambda i: (i, 0)),
            pl.BlockSpec((_BN, _H), lambda i: (i, 0)),
        ],
        out_shape=[
            jax.ShapeDtypeStruct((_N, _HE), jnp.float32),
            jax.ShapeDtypeStruct((_N, _H), jnp.float32),
        ],
    )(x, w_in, b_in, w_r, b_l)


def _sc_body(feat_hbm, srcs_hbm, dsts_hbm, out_hbm,
             src_v, dst_v, buf0, buf1, acc_sh, sem0, sem1, isem):
    c = lax.axis_index("c")
    s = lax.axis_index("s")
    w = s * _NC + c

    icp0 = pltpu.make_async_copy(srcs_hbm.at[w], src_v, isem)
    icp0.start()
    icp1 = pltpu.make_async_copy(dsts_hbm.at[w], dst_v, isem)
    icp1.start()

    def zrow(i, _):
        def zcol(j, _):
            buf0[i, pl.ds(j * 16, 16)] = jnp.zeros((16,), jnp.float32)
            return 0
        return lax.fori_loop(0, _HE // 16, zcol, 0)
    lax.fori_loop(0, _CHUNK, zrow, 0)

    def zcp(t, _):
        pltpu.sync_copy(buf0,
                        acc_sh.at[pl.ds(s * _RPS + t * _CHUNK, _CHUNK)])
        return 0
    lax.fori_loop(0, _RPS // _CHUNK, zcp, 0)

    ztail = _RPS - (_RPS // _CHUNK) * _CHUNK
    def zcp8(t, _):
        pltpu.sync_copy(
            buf0.at[pl.ds(0, 8)],
            acc_sh.at[pl.ds(s * _RPS + (_RPS // _CHUNK) * _CHUNK + t * 8, 8)])
        return 0
    lax.fori_loop(0, ztail // 8, zcp8, 0)
    icp0.wait()
    icp1.wait()
    plsc.subcore_barrier()

    def g_start(j, buf, sem):
        pltpu.make_async_copy(feat_hbm.at[src_v.at[j]], buf, sem).start()

    def g_wait(j, buf, sem):
        pltpu.make_async_copy(feat_hbm.at[src_v.at[j]], buf, sem).wait()

    g_start(0, buf0, sem0)

    def pair(i, _):
        a = 2 * i
        b = a + 1
        g_start(b, buf1, sem1)
        g_wait(a, buf0, sem0)
        pltpu.sync_copy(buf0, acc_sh.at[dst_v.at[a]], add=True)
        g_start(a + 2, buf0, sem0)
        g_wait(b, buf1, sem1)
        pltpu.sync_copy(buf1, acc_sh.at[dst_v.at[b]], add=True)
        return 0
    lax.fori_loop(0, _NCHUNK // 2, pair, 0)
    g_wait(_NCHUNK, buf0, sem0)
    plsc.subcore_barrier()

    base = s * _RPS
    pltpu.sync_copy(acc_sh.at[pl.ds(base, _RPS)],
                    out_hbm.at[c, pl.ds(base, _RPS)])


_sc_agg = functools.partial(
    pl.kernel,
    out_type=jax.ShapeDtypeStruct((_NC, _AROWS, _HE), jnp.float32),
    mesh=plsc.VectorSubcoreMesh(core_axis_name="c", subcore_axis_name="s"),
    compiler_params=pltpu.CompilerParams(use_tc_tiling_on_sc=False),
    scratch_types=[
        pltpu.VMEM((_NCHUNKA, _CHUNK), jnp.int32),
        pltpu.VMEM((_NCHUNKA, _CHUNK), jnp.int32),
        pltpu.VMEM((_CHUNK, _HE), jnp.float32),
        pltpu.VMEM((_CHUNK, _HE), jnp.float32),
        pltpu.VMEM_SHARED((_AROWS, _HE), jnp.float32),
        pltpu.SemaphoreType.DMA,
        pltpu.SemaphoreType.DMA,
        pltpu.SemaphoreType.DMA,
    ],
)(_sc_body)


def _post_body(p0_ref, p1_ref, base_ref, wl_ref, wcls_ref, out_ref, of_ref):
    acc = p0_ref[:, :_H] + p1_ref[:, :_H]
    cnt = p0_ref[:, _H:_H + 1] + p1_ref[:, _H:_H + 1]
    mean = acc / jnp.maximum(cnt, 1.0)
    of = jnp.dot(mean, wl_ref[...], preferred_element_type=jnp.float32)
    of = of + base_ref[...]
    of_ref[...] = of
    nrm = jnp.sqrt(jnp.sum(of * of, axis=1, keepdims=True))
    xn = of / jnp.maximum(nrm, 1e-12)
    wc = wcls_ref[...]
    wnrm = jnp.sqrt(jnp.sum(wc * wc, axis=0, keepdims=True))
    wn = wc / jnp.maximum(wnrm, 1e-12)
    out_ref[...] = _TEMP * jnp.dot(xn, wn, preferred_element_type=jnp.float32)


def _post(p0, p1, base, w_l, w_cls):
    grid = _N // _BN
    return pl.pallas_call(
        _post_body,
        grid=(grid,),
        in_specs=[
            pl.BlockSpec((_BN, _HE), lambda i: (i, 0)),
            pl.BlockSpec((_BN, _HE), lambda i: (i, 0)),
            pl.BlockSpec((_BN, _H), lambda i: (i, 0)),
            pl.BlockSpec((_H, _H), lambda i: (0, 0)),
            pl.BlockSpec((_H, _C), lambda i: (0, 0)),
        ],
        out_specs=[
            pl.BlockSpec((_BN, _C), lambda i: (i, 0)),
            pl.BlockSpec((_BN, _H), lambda i: (i, 0)),
        ],
        out_shape=[
            jax.ShapeDtypeStruct((_N, _C), jnp.float32),
            jax.ShapeDtypeStruct((_N, _H), jnp.float32),
        ],
    )(p0, p1, base, w_l, w_cls)


def kernel(x, edge_index, W_in, b_in, W_l, b_l, W_r, W_cls):
    feat_ext, base = _pre(x, W_in, b_in.reshape(1, _H),
                          W_r, b_l.reshape(1, _H))

    pad = _EPAD - _E
    dummy_dst = (_N + (jnp.arange(pad, dtype=jnp.int32) % (_AROWS - _N)))
    src = jnp.concatenate([edge_index[0], jnp.zeros((pad,), jnp.int32)])
    dst = jnp.concatenate([edge_index[1], dummy_dst])
    srcs = src.reshape(_NCHUNK, _NW, _CHUNK).transpose(1, 0, 2)
    dsts = dst.reshape(_NCHUNK, _NW, _CHUNK).transpose(1, 0, 2)
    zc = jnp.zeros((_NW, 1, _CHUNK), jnp.int32)
    srcs = jnp.concatenate([srcs, zc], axis=1)
    dsts = jnp.concatenate([dsts, zc + _N], axis=1)

    parts = _sc_agg(feat_ext, srcs, dsts)

    out, out_feat = _post(parts[0, :_N], parts[1, :_N], base, W_l, W_cls)
    return (out, out_feat)


# speedup vs baseline: 1.0474x; 1.0474x over previous
import functools

import jax
import jax.numpy as jnp
from jax import lax
from jax.experimental import pallas as pl
from jax.experimental.pallas import tpu as pltpu
from jax.experimental.pallas import tpu_sc as plsc

_N = 10000
_E = 320000
_D = 128
_H = 128
_C = 20
_TEMP = 10.0

_HE = 144
_NC = 2
_NS = 16
_NW = _NC * _NS
_CHUNK = 128
_NCHUNK = 80
_NCHUNKA = _NCHUNK + 1
_EPAD = _NW * _NCHUNK * _CHUNK
_RPS = 632
_AROWS = _NS * _RPS

_BN = 2000


def _pre_body(x_ref, win_ref, bin_ref, wr_ref, bl_ref, fe_ref, base_ref):
    xb = x_ref[...]
    feat = jnp.dot(xb, win_ref[...], preferred_element_type=jnp.float32)
    feat = jnp.maximum(feat + bin_ref[...], 0.0)
    col = lax.broadcasted_iota(jnp.int32, (_BN, _HE - _H), 1)
    tail = jnp.where(col == 0, 1.0, 0.0).astype(jnp.float32)
    fe_ref[...] = jnp.concatenate([feat, tail], axis=1)
    base = jnp.dot(feat, wr_ref[...], preferred_element_type=jnp.float32)
    base_ref[...] = base + bl_ref[...]


def _pre(x, w_in, b_in, w_r, b_l):
    grid = _N // _BN
    return pl.pallas_call(
        _pre_body,
        grid=(grid,),
        in_specs=[
            pl.BlockSpec((_BN, _D), lambda i: (i, 0)),
            pl.BlockSpec((_D, _H), lambda i: (0, 0)),
            pl.BlockSpec((1, _H), lambda i: (0, 0)),
            pl.BlockSpec((_H, _H), lambda i: (0, 0)),
            pl.BlockSpec((1, _H), lambda i: (0, 0)),
        ],
        out_specs=[
            pl.BlockSpec((_BN, _HE), lambda i: (i, 0)),
            pl.BlockSpec((_BN, _H), lambda i: (i, 0)),
        ],
        out_shape=[
            jax.ShapeDtypeStruct((_N, _HE), jnp.float32),
            jax.ShapeDtypeStruct((_N, _H), jnp.float32),
        ],
    )(x, w_in, b_in, w_r, b_l)


def _sc_body(feat_hbm, srcs_hbm, dsts_hbm, out_hbm,
             src_v, dst_v, buf0, acc_sh, sem0, isem):
    c = lax.axis_index("c")
    s = lax.axis_index("s")
    w = s * _NC + c

    icp0 = pltpu.make_async_copy(srcs_hbm.at[w], src_v, isem)
    icp0.start()
    icp1 = pltpu.make_async_copy(dsts_hbm.at[w], dst_v, isem)
    icp1.start()

    def zrow(i, _):
        def zcol(j, _):
            buf0[i, pl.ds(j * 16, 16)] = jnp.zeros((16,), jnp.float32)
            return 0
        return lax.fori_loop(0, _HE // 16, zcol, 0)
    lax.fori_loop(0, _CHUNK, zrow, 0)

    def zcp(t, _):
        pltpu.sync_copy(buf0,
                        acc_sh.at[pl.ds(s * _RPS + t * _CHUNK, _CHUNK)])
        return 0
    lax.fori_loop(0, _RPS // _CHUNK, zcp, 0)

    ztail = _RPS - (_RPS // _CHUNK) * _CHUNK
    def zcp8(t, _):
        pltpu.sync_copy(
            buf0.at[pl.ds(0, 8)],
            acc_sh.at[pl.ds(s * _RPS + (_RPS // _CHUNK) * _CHUNK + t * 8, 8)])
        return 0
    lax.fori_loop(0, ztail // 8, zcp8, 0)
    icp0.wait()
    icp1.wait()
    plsc.subcore_barrier()

    def chunk(j, _):
        pltpu.async_copy(feat_hbm.at[src_v.at[j]], buf0, sem0).wait()
        pltpu.sync_copy(buf0, acc_sh.at[dst_v.at[j]], add=True)
        return 0
    lax.fori_loop(0, _NCHUNK, chunk, 0)
    plsc.subcore_barrier()

    base = s * _RPS
    pltpu.sync_copy(acc_sh.at[pl.ds(base, _RPS)],
                    out_hbm.at[c, pl.ds(base, _RPS)])


_sc_agg = functools.partial(
    pl.kernel,
    out_type=jax.ShapeDtypeStruct((_NC, _AROWS, _HE), jnp.float32),
    mesh=plsc.VectorSubcoreMesh(core_axis_name="c", subcore_axis_name="s"),
    compiler_params=pltpu.CompilerParams(use_tc_tiling_on_sc=False),
    scratch_types=[
        pltpu.VMEM((_NCHUNKA, _CHUNK), jnp.int32),
        pltpu.VMEM((_NCHUNKA, _CHUNK), jnp.int32),
        pltpu.VMEM((_CHUNK, _HE), jnp.float32),
        pltpu.VMEM_SHARED((_AROWS, _HE), jnp.float32),
        pltpu.SemaphoreType.DMA,
        pltpu.SemaphoreType.DMA,
    ],
)(_sc_body)


def _post_body(p0_ref, p1_ref, base_ref, wl_ref, wcls_ref, out_ref, of_ref):
    acc = p0_ref[:, :_H] + p1_ref[:, :_H]
    cnt = p0_ref[:, _H:_H + 1] + p1_ref[:, _H:_H + 1]
    mean = acc / jnp.maximum(cnt, 1.0)
    of = jnp.dot(mean, wl_ref[...], preferred_element_type=jnp.float32)
    of = of + base_ref[...]
    of_ref[...] = of
    nrm = jnp.sqrt(jnp.sum(of * of, axis=1, keepdims=True))
    xn = of / jnp.maximum(nrm, 1e-12)
    wc = wcls_ref[...]
    wnrm = jnp.sqrt(jnp.sum(wc * wc, axis=0, keepdims=True))
    wn = wc / jnp.maximum(wnrm, 1e-12)
    out_ref[...] = _TEMP * jnp.dot(xn, wn, preferred_element_type=jnp.float32)


def _post(p0, p1, base, w_l, w_cls):
    grid = _N // _BN
    return pl.pallas_call(
        _post_body,
        grid=(grid,),
        in_specs=[
            pl.BlockSpec((_BN, _HE), lambda i: (i, 0)),
            pl.BlockSpec((_BN, _HE), lambda i: (i, 0)),
            pl.BlockSpec((_BN, _H), lambda i: (i, 0)),
            pl.BlockSpec((_H, _H), lambda i: (0, 0)),
            pl.BlockSpec((_H, _C), lambda i: (0, 0)),
        ],
        out_specs=[
            pl.BlockSpec((_BN, _C), lambda i: (i, 0)),
            pl.BlockSpec((_BN, _H), lambda i: (i, 0)),
        ],
        out_shape=[
            jax.ShapeDtypeStruct((_N, _C), jnp.float32),
            jax.ShapeDtypeStruct((_N, _H), jnp.float32),
        ],
    )(p0, p1, base, w_l, w_cls)


def kernel(x, edge_index, W_in, b_in, W_l, b_l, W_r, W_cls):
    feat_ext, base = _pre(x, W_in, b_in.reshape(1, _H),
                          W_r, b_l.reshape(1, _H))

    pad = _EPAD - _E
    dummy_dst = (_N + (jnp.arange(pad, dtype=jnp.int32) % (_AROWS - _N)))
    src = jnp.concatenate([edge_index[0], jnp.zeros((pad,), jnp.int32)])
    dst = jnp.concatenate([edge_index[1], dummy_dst])
    srcs = src.reshape(_NCHUNK, _NW, _CHUNK).transpose(1, 0, 2)
    dsts = dst.reshape(_NCHUNK, _NW, _CHUNK).transpose(1, 0, 2)
    zc = jnp.zeros((_NW, 1, _CHUNK), jnp.int32)
    srcs = jnp.concatenate([srcs, zc], axis=1)
    dsts = jnp.concatenate([dsts, zc + _N], axis=1)

    parts = _sc_agg(feat_ext, srcs, dsts)

    out, out_feat = _post(parts[0, :_N], parts[1, :_N], base, W_l, W_cls)
    return (out, out_feat)
